# baseline (speedup 1.0000x reference)
import functools
import math

import jax
import jax.numpy as jnp
from jax import lax
from jax.experimental import pallas as pl
from jax.experimental.pallas import tpu as pltpu
from jax.experimental.pallas import tpu_sc as plsc

D_MODEL = 128
SBLK = 256
LANES = 16


@functools.cache
def _make_gather(n_total: int, vocab: int, d: int, nc: int, ns: int):
    nw = nc * ns
    r_per_w = n_total // nw
    n_sg = r_per_w // SBLK
    mesh = plsc.VectorSubcoreMesh(core_axis_name="c", subcore_axis_name="s")

    NBUF = 3
    LOOKAHEAD = 1
    padded = ((n_sg + NBUF - 1) // NBUF) * NBUF

    @functools.partial(
        pl.kernel,
        out_type=jax.ShapeDtypeStruct((n_total, d), jnp.float32),
        mesh=mesh,
        scratch_types=[
            pltpu.VMEM((r_per_w,), jnp.int32),
            pltpu.VMEM((NBUF, SBLK, d), jnp.float32),
            [pltpu.SemaphoreType.DMA] * NBUF,
            [pltpu.SemaphoreType.DMA] * NBUF,
        ],
    )
    def gather_kernel(idx_hbm, lut_hbm, out_hbm, idx_v, rows_v, gsem, wsem):
        cid = lax.axis_index("c")
        sid = lax.axis_index("s")
        wid = sid * nc + cid
        r_base = wid * r_per_w

        pltpu.sync_copy(idx_hbm.at[wid], idx_v)

        scale = jnp.float32(math.sqrt(d))
        vecs_per_row = d // LANES

        def fire_gather(sg, slot):
            pltpu.async_copy(
                lut_hbm.at[idx_v.at[pl.ds(sg * SBLK, SBLK)]], rows_v.at[slot],
                gsem[slot])

        def out_copy(sg, slot, sem):
            return pltpu.make_async_copy(
                rows_v.at[slot],
                out_hbm.at[pl.ds(r_base + sg * SBLK, SBLK)], sem)

        for sg in range(LOOKAHEAD):
            fire_gather(sg, sg % NBUF)

        @pl.loop(0, padded, step=NBUF)
        def _ring(j0):
            for b in range(NBUF):
                sg = j0 + b

                @pl.when(sg < n_sg)
                def _process():
                    pltpu.make_async_copy(
                        lut_hbm.at[idx_v.at[pl.ds(sg * SBLK, SBLK)]],
                        rows_v.at[b], gsem[b]).wait()

                    nslot = (b + LOOKAHEAD) % NBUF
                    sgn = sg + LOOKAHEAD
                    sgo = sg - (NBUF - LOOKAHEAD)

                    @pl.when(jnp.logical_and(sgo >= 0, sgn < n_sg))
                    def _drain_old_write():
                        out_copy(sgo, nslot, wsem[nslot]).wait()

                    @pl.when(sgn < n_sg)
                    def _refill():
                        fire_gather(sgn, nslot)

                    @pl.loop(0, SBLK)
                    def _row(r):
                        for k in range(vecs_per_row):
                            sl = pl.ds(k * LANES, LANES)
                            rows_v[b, r, sl] = rows_v[b, r, sl] * scale

                    out_copy(sg, b, wsem[b]).start()

        for k in range(NBUF):
            sg = n_sg - NBUF + k
            out_copy(sg, sg % NBUF, wsem[sg % NBUF]).wait()

    return gather_kernel


def kernel(x, lut):
    vocab, d = lut.shape
    n = x.size
    info = plsc.get_sparse_core_info()
    nc, ns = info.num_cores, info.num_subcores
    nw = nc * ns
    assert n % (nw * SBLK) == 0
    idx2d = x.reshape(nw, n // nw).astype(jnp.int32)
    out = _make_gather(n, vocab, d, nc, ns)(idx2d, lut)
    return out.reshape(x.shape + (d,))


# speedup vs baseline: 1.0325x; 1.0325x over previous
import functools
import math

import jax
import jax.numpy as jnp
from jax import lax
from jax.experimental import pallas as pl
from jax.experimental.pallas import tpu as pltpu
from jax.experimental.pallas import tpu_sc as plsc

D_MODEL = 128
SBLK = 128
LANES = 16


@functools.cache
def _make_gather(n_total: int, vocab: int, d: int, nc: int, ns: int):
    nw = nc * ns
    r_per_w = n_total // nw
    n_sg = r_per_w // SBLK
    mesh = plsc.VectorSubcoreMesh(core_axis_name="c", subcore_axis_name="s")

    NBUF = 7
    LOOKAHEAD = 5
    padded = ((n_sg + NBUF - 1) // NBUF) * NBUF

    @functools.partial(
        pl.kernel,
        out_type=jax.ShapeDtypeStruct((n_total, d), jnp.float32),
        mesh=mesh,
        scratch_types=[
            pltpu.VMEM((r_per_w,), jnp.int32),
            pltpu.VMEM((NBUF, SBLK, d), jnp.float32),
            [pltpu.SemaphoreType.DMA] * NBUF,
            [pltpu.SemaphoreType.DMA] * NBUF,
        ],
    )
    def gather_kernel(idx_hbm, lut_hbm, out_hbm, idx_v, rows_v, gsem, wsem):
        cid = lax.axis_index("c")
        sid = lax.axis_index("s")
        wid = sid * nc + cid
        r_base = wid * r_per_w

        pltpu.sync_copy(idx_hbm.at[wid], idx_v)

        scale = jnp.float32(math.sqrt(d))
        vecs_per_row = d // LANES

        def fire_gather(sg, slot):
            pltpu.async_copy(
                lut_hbm.at[idx_v.at[pl.ds(sg * SBLK, SBLK)]],
                rows_v.at[slot], gsem[slot])

        def out_copy(sg, slot, sem):
            return pltpu.make_async_copy(
                rows_v.at[slot],
                out_hbm.at[pl.ds(r_base + sg * SBLK, SBLK)], sem)

        for sg in range(LOOKAHEAD):
            fire_gather(sg, sg % NBUF)

        @pl.loop(0, padded, step=NBUF)
        def _ring(j0):
            for b in range(NBUF):
                sg = j0 + b

                @pl.when(sg < n_sg)
                def _process():
                    pltpu.make_async_copy(
                        lut_hbm.at[idx_v.at[pl.ds(sg * SBLK, SBLK)]],
                        rows_v.at[b], gsem[b]).wait()

                    nslot = (b + LOOKAHEAD) % NBUF
                    sgn = sg + LOOKAHEAD
                    sgo = sg - (NBUF - LOOKAHEAD)

                    @pl.when(jnp.logical_and(sgo >= 0, sgn < n_sg))
                    def _drain_old_write():
                        out_copy(sgo, nslot, wsem[nslot]).wait()

                    @pl.when(sgn < n_sg)
                    def _refill():
                        fire_gather(sgn, nslot)

                    @pl.loop(0, SBLK)
                    def _row(r):
                        for k in range(vecs_per_row):
                            sl = pl.ds(k * LANES, LANES)
                            rows_v[b, r, sl] = rows_v[b, r, sl] * scale

                    out_copy(sg, b, wsem[b]).start()

        for k in range(NBUF):
            sg = n_sg - NBUF + k
            out_copy(sg, sg % NBUF, wsem[sg % NBUF]).wait()

    return gather_kernel


def kernel(x, lut):
    vocab, d = lut.shape
    n = x.size
    info = plsc.get_sparse_core_info()
    nc, ns = info.num_cores, info.num_subcores
    nw = nc * ns
    assert n % (nw * SBLK) == 0
    idx2d = x.reshape(nw, n // nw).astype(jnp.int32)
    out = _make_gather(n, vocab, d, nc, ns)(idx2d, lut)
    return out.reshape(x.shape + (d,))


# speedup vs baseline: 1.0330x; 1.0005x over previous
import functools
import math

import jax
import jax.numpy as jnp
from jax import lax
from jax.experimental import pallas as pl
from jax.experimental.pallas import tpu as pltpu
from jax.experimental.pallas import tpu_sc as plsc

D_MODEL = 128
SBLK = 128
LANES = 16


@functools.cache
def _make_gather(n_total: int, vocab: int, d: int, nc: int, ns: int):
    nw = nc * ns
    r_per_w = n_total // nw
    n_sg = r_per_w // SBLK
    mesh = plsc.VectorSubcoreMesh(core_axis_name="c", subcore_axis_name="s")

    NBUF = 7
    LOOKAHEAD = 6
    padded = ((n_sg + NBUF - 1) // NBUF) * NBUF

    @functools.partial(
        pl.kernel,
        out_type=jax.ShapeDtypeStruct((n_total, d), jnp.float32),
        mesh=mesh,
        scratch_types=[
            pltpu.VMEM((r_per_w,), jnp.int32),
            pltpu.VMEM((NBUF, SBLK, d), jnp.float32),
            [pltpu.SemaphoreType.DMA] * NBUF,
            [pltpu.SemaphoreType.DMA] * NBUF,
        ],
    )
    def gather_kernel(idx_hbm, lut_hbm, out_hbm, idx_v, rows_v, gsem, wsem):
        cid = lax.axis_index("c")
        sid = lax.axis_index("s")
        wid = sid * nc + cid
        r_base = wid * r_per_w

        pltpu.sync_copy(idx_hbm.at[wid], idx_v)

        scale = jnp.float32(math.sqrt(d))
        vecs_per_row = d // LANES

        def fire_gather(sg, slot):
            pltpu.async_copy(
                lut_hbm.at[idx_v.at[pl.ds(sg * SBLK, SBLK)]],
                rows_v.at[slot], gsem[slot])

        def out_copy(sg, slot, sem):
            return pltpu.make_async_copy(
                rows_v.at[slot],
                out_hbm.at[pl.ds(r_base + sg * SBLK, SBLK)], sem)

        for sg in range(LOOKAHEAD):
            fire_gather(sg, sg % NBUF)

        @pl.loop(0, padded, step=NBUF)
        def _ring(j0):
            for b in range(NBUF):
                sg = j0 + b

                @pl.when(sg < n_sg)
                def _process():
                    pltpu.make_async_copy(
                        lut_hbm.at[idx_v.at[pl.ds(sg * SBLK, SBLK)]],
                        rows_v.at[b], gsem[b]).wait()

                    nslot = (b + LOOKAHEAD) % NBUF
                    sgn = sg + LOOKAHEAD
                    sgo = sg - (NBUF - LOOKAHEAD)

                    @pl.when(jnp.logical_and(sgo >= 0, sgn < n_sg))
                    def _drain_old_write():
                        out_copy(sgo, nslot, wsem[nslot]).wait()

                    @pl.when(sgn < n_sg)
                    def _refill():
                        fire_gather(sgn, nslot)

                    @pl.loop(0, SBLK)
                    def _row(r):
                        for k in range(vecs_per_row):
                            sl = pl.ds(k * LANES, LANES)
                            rows_v[b, r, sl] = rows_v[b, r, sl] * scale

                    out_copy(sg, b, wsem[b]).start()

        for k in range(NBUF):
            sg = n_sg - NBUF + k
            out_copy(sg, sg % NBUF, wsem[sg % NBUF]).wait()

    return gather_kernel


def kernel(x, lut):
    vocab, d = lut.shape
    n = x.size
    info = plsc.get_sparse_core_info()
    nc, ns = info.num_cores, info.num_subcores
    nw = nc * ns
    assert n % (nw * SBLK) == 0
    idx2d = x.reshape(nw, n // nw).astype(jnp.int32)
    out = _make_gather(n, vocab, d, nc, ns)(idx2d, lut)
    return out.reshape(x.shape + (d,))


# speedup vs baseline: 1.0334x; 1.0004x over previous
import functools
import math

import jax
import jax.numpy as jnp
from jax import lax
from jax.experimental import pallas as pl
from jax.experimental.pallas import tpu as pltpu
from jax.experimental.pallas import tpu_sc as plsc

D_MODEL = 128
SBLK = 128
LANES = 16


@functools.cache
def _make_gather(n_total: int, vocab: int, d: int, nc: int, ns: int):
    nw = nc * ns
    r_per_w = n_total // nw
    n_sg = r_per_w // SBLK
    mesh = plsc.VectorSubcoreMesh(core_axis_name="c", subcore_axis_name="s")

    NBUF = 7
    LOOKAHEAD = 5
    padded = ((n_sg + NBUF - 1) // NBUF) * NBUF

    @functools.partial(
        pl.kernel,
        out_type=jax.ShapeDtypeStruct((n_total, d), jnp.float32),
        mesh=mesh,
        scratch_types=[
            pltpu.VMEM((r_per_w,), jnp.int32),
            pltpu.VMEM((NBUF, SBLK, d), jnp.float32),
            [pltpu.SemaphoreType.DMA] * NBUF,
            [pltpu.SemaphoreType.DMA] * NBUF,
        ],
    )
    def gather_kernel(idx_hbm, lut_hbm, out_hbm, idx_v, rows_v, gsem, wsem):
        cid = lax.axis_index("c")
        sid = lax.axis_index("s")
        wid = sid * nc + cid
        r_base = wid * r_per_w

        pltpu.sync_copy(idx_hbm.at[wid], idx_v)

        scale = jnp.float32(math.sqrt(d))
        vecs_per_row = d // LANES

        def fire_gather(sg, slot):
            pltpu.async_copy(
                lut_hbm.at[idx_v.at[pl.ds(sg * SBLK, SBLK)]],
                rows_v.at[slot], gsem[slot])

        def out_copy(sg, slot, sem):
            return pltpu.make_async_copy(
                rows_v.at[slot],
                out_hbm.at[pl.ds(r_base + sg * SBLK, SBLK)], sem)

        for sg in range(LOOKAHEAD):
            fire_gather(sg, sg % NBUF)

        @pl.loop(0, padded, step=NBUF)
        def _ring(j0):
            for b in range(NBUF):
                sg = j0 + b

                @pl.when(sg < n_sg)
                def _process():
                    pltpu.make_async_copy(
                        lut_hbm.at[idx_v.at[pl.ds(sg * SBLK, SBLK)]],
                        rows_v.at[b], gsem[b]).wait()

                    nslot = (b + LOOKAHEAD) % NBUF
                    sgn = sg + LOOKAHEAD
                    sgo = sg - (NBUF - LOOKAHEAD)

                    @pl.when(jnp.logical_and(sgo >= 0, sgn < n_sg))
                    def _drain_old_write():
                        out_copy(sgo, nslot, wsem[nslot]).wait()

                    @pl.when(sgn < n_sg)
                    def _refill():
                        fire_gather(sgn, nslot)

                    @pl.loop(0, SBLK)
                    def _row(r):
                        for k in range(vecs_per_row):
                            sl = pl.ds(k * LANES, LANES)
                            rows_v[b, r, sl] = rows_v[b, r, sl] * scale

                    out_copy(sg, b, wsem[b]).start()

        for k in range(NBUF):
            sg = n_sg - NBUF + k
            out_copy(sg, sg % NBUF, wsem[sg % NBUF]).wait()

    return gather_kernel


def kernel(x, lut):
    vocab, d = lut.shape
    n = x.size
    info = plsc.get_sparse_core_info()
    nc, ns = info.num_cores, info.num_subcores
    nw = nc * ns
    assert n % (nw * SBLK) == 0
    idx2d = x.reshape(nw, n // nw).astype(jnp.int32)
    out = _make_gather(n, vocab, d, nc, ns)(idx2d, lut)
    return out.reshape(x.shape + (d,))
